# baseline (speedup 1.0000x reference)
import functools

import numpy as np
import jax
import jax.numpy as jnp
from jax import lax
from jax.experimental import pallas as pl
from jax.experimental.pallas import tpu as pltpu
from jax.experimental.pallas import tpu_sc as plsc

_TIME_RES = 25
_N_LEVELS = 8
_F = 4
_BASE_RES = 512
_MAX_RES = 32768
_HASHMAP = 1 << 16
_PER_LEVEL_SCALE = float(np.exp2(np.log2(_MAX_RES / _BASE_RES) / (_N_LEVELS - 1)))
_PRIME = np.uint32(2654435761).astype(np.int32)
_SCALES = [np.float32(_BASE_RES * (_PER_LEVEL_SCALE ** l) - 1.0) for l in range(_N_LEVELS)]

_N = 1048576
_NW = 32
_PTS = _N // _NW
_B = 1024
_NCH = _PTS // _B
_GROUPS_A = _B // 16
_GROUPS_B = _B // 4


def _combine_sc_body(tn_hbm, sel_hbm, wb_hbm, c_hbm, t1_v, t2_v, o_v, w_v, sel_v):
    cid = lax.axis_index("c")
    sid = lax.axis_index("s")
    wid = sid * 2 + cid

    i16 = lax.iota(jnp.int32, 16)
    i8 = i16 * 8
    zero16 = jnp.zeros((16,), jnp.float32)

    pltpu.sync_copy(wb_hbm, w_v)
    pltpu.sync_copy(sel_hbm, sel_v)
    w1v = w_v[pl.ds(0, 16)]
    w2v = w_v[pl.ds(16, 16)]
    selvec = sel_v[pl.ds(0, 16)]
    sbase1 = selvec[0] * 2097152
    sbase2 = selvec[1] * 2097152

    def z(g, _):
        o_v[pl.ds(g * 16, 16)] = zero16
        return _

    lax.fori_loop(0, 1024, z, None)

    def chunk(ch, _):
        base = wid * 65536 + ch * 8192
        pltpu.sync_copy(tn_hbm.at[pl.ds(sbase1 + base, 8192)], t1_v)
        pltpu.sync_copy(tn_hbm.at[pl.ds(sbase2 + base, 8192)], t2_v)

        def blk(k, _):
            ib = (k >> 3) * 512 + (k & 7) * 16
            ob = (k >> 3) * 1024 + (k & 7) * 128
            for f in range(4):
                v = w1v * t1_v[pl.ds(ib + f * 128, 16)] + w2v * t2_v[pl.ds(ib + f * 128, 16)]
                plsc.store_scatter(o_v, [ob + i8 + f], v)
            return _

        lax.fori_loop(0, 128, blk, None)
        pltpu.sync_copy(o_v, c_hbm.at[pl.ds(wid * 131072 + ch * 16384, 16384)])
        return _

    lax.fori_loop(0, 8, chunk, None)


@functools.cache
def _get_sc_combine():
    return functools.partial(
        pl.kernel,
        out_type=jax.ShapeDtypeStruct((_N_LEVELS * _HASHMAP * 8,), jnp.float32),
        mesh=plsc.VectorSubcoreMesh(
            core_axis_name="c", subcore_axis_name="s", num_cores=2, num_subcores=16
        ),
        compiler_params=pltpu.CompilerParams(
            needs_layout_passes=False, use_tc_tiling_on_sc=False
        ),
        scratch_types=[
            pltpu.VMEM((8192,), jnp.float32),
            pltpu.VMEM((8192,), jnp.float32),
            pltpu.VMEM((16384,), jnp.float32),
            pltpu.VMEM((32,), jnp.float32),
            pltpu.VMEM((16,), jnp.int32),
        ],
    )(_combine_sc_body)


def _sc_body(
    c_hbm, x_hbm, out_hbm, x_v, ih_v, f_v, rows_v, out_v, sem0, sem1, xsem, osem
):
    cid = lax.axis_index("c")
    sid = lax.axis_index("s")
    wid = sid * 2 + cid
    base_pt = wid * _PTS

    i16 = lax.iota(jnp.int32, 16)
    fcols = tuple(i16 * 0 + f for f in range(4))

    sems = (sem0, sem1)

    def x_dma(ci, xp):
        return pltpu.make_async_copy(
            x_hbm.at[pl.ds((base_pt + ci * _B) * 2, _B * 2)], x_v.at[xp], xsem
        )

    def out_dma(ci, fg):
        px0 = base_pt + ci * _B
        return pltpu.make_async_copy(
            out_v.at[pl.ds(fg * 8192, 8192)],
            out_hbm.at[pl.ds(fg * 8388608 + px0 * 8, 8192)],
            osem,
        )

    def pass_a(lvl, parity, xp):
        scale = _SCALES[lvl]
        fbase = parity * 2048
        ihbase = parity * 32
        lb = np.int32(lvl * _HASHMAP)

        def ga(g, _):
            xb = (g >> 3) * 256 + (g & 7) * 16
            xs = x_v[xp, pl.ds(xb, 16)]
            ys = x_v[xp, pl.ds(xb + 128, 16)]
            pxf = xs * scale + 0.5
            pyf = ys * scale + 0.5
            pxi = pxf.astype(jnp.int32)
            pyi = pyf.astype(jnp.int32)
            f_v[pl.ds(fbase + g * 16, 16)] = pxf - pxi.astype(jnp.float32)
            f_v[pl.ds(fbase + 1024 + g * 16, 16)] = pyf - pyi.astype(jnp.float32)
            m0 = pyi * _PRIME
            m1 = m0 + _PRIME
            px1 = pxi + 1
            row = ihbase + (g >> 3)
            col = (g & 7) * 16
            ih_v[row, pl.ds(col, 16)] = ((pxi ^ m0) & 65535) + lb
            ih_v[row + 8, pl.ds(col, 16)] = ((px1 ^ m0) & 65535) + lb
            ih_v[row + 16, pl.ds(col, 16)] = ((pxi ^ m1) & 65535) + lb
            ih_v[row + 24, pl.ds(col, 16)] = ((px1 ^ m1) & 65535) + lb
            return _

        lax.fori_loop(0, _GROUPS_A, ga, None)

    def fire(parity):
        sem = sems[parity]

        def fj(r, _):
            src = c_hbm.at[ih_v.at[parity * 32 + r]]
            dst = rows_v.at[pl.ds(parity * 4096 + r * 128, 128)]
            pltpu.async_copy(src, dst, sem)
            return _

        lax.fori_loop(0, 32, fj, None)

    def drain(parity):
        sem = sems[parity]

        def dj(r, _):
            src = c_hbm.at[ih_v.at[parity * 32 + r]]
            dst = rows_v.at[pl.ds(parity * 4096 + r * 128, 128)]
            pltpu.make_async_copy(src, dst, sem).wait()
            return _

        lax.fori_loop(0, 32, dj, None)

    def pass_b(lvl, parity):
        rbase = parity * 4096
        fbase = parity * 2048
        obase = np.int32((lvl >> 1) * 8192 + (lvl & 1) * 512)

        def gb(q, _):
            qb = q * 16
            fx = f_v[pl.ds(fbase + qb, 16)]
            fy = f_v[pl.ds(fbase + 1024 + qb, 16)]
            gx0 = 1.0 - fx
            gy0 = 1.0 - fy
            ridx0 = rbase + qb + i16
            ridx1 = ridx0 + 1024
            ridx2 = ridx0 + 2048
            ridx3 = ridx0 + 3072
            ob = obase + (q >> 3) * 1024 + (q & 7) * 16
            for f in range(4):
                r0 = plsc.load_gather(rows_v, [ridx0, fcols[f]])
                r1 = plsc.load_gather(rows_v, [ridx1, fcols[f]])
                r2 = plsc.load_gather(rows_v, [ridx2, fcols[f]])
                r3 = plsc.load_gather(rows_v, [ridx3, fcols[f]])
                acc = gy0 * (gx0 * r0 + fx * r1) + fy * (gx0 * r2 + fx * r3)
                out_v[pl.ds(ob + f * 128, 16)] = acc
            return _

        lax.fori_loop(0, _GROUPS_A, gb, None)

    def chunk(ci, _):
        xp = ci & 1
        x_dma(ci, xp).wait()

        @pl.when(ci < _NCH - 1)
        def _():
            x_dma(ci + 1, 1 - xp).start()

        pass_a(0, 0, xp)
        fire(0)
        pass_a(1, 1, xp)
        fire(1)

        @pl.when(ci > 0)
        def _():
            for fg in range(4):
                out_dma(ci - 1, fg).wait()

        drain(0)
        pass_b(0, 0)
        for lvl in range(2, _N_LEVELS):
            pass_a(lvl, lvl % 2, xp)
            fire(lvl % 2)
            drain((lvl - 1) % 2)
            pass_b(lvl - 1, (lvl - 1) % 2)
        drain((_N_LEVELS - 1) % 2)
        pass_b(_N_LEVELS - 1, (_N_LEVELS - 1) % 2)
        for fg in range(4):
            out_dma(ci, fg).start()
        return _

    x_dma(0, 0).start()
    lax.fori_loop(0, _NCH, chunk, None)
    for fg in range(4):
        out_dma(_NCH - 1, fg).wait()


@functools.cache
def _get_sc_encode():
    return functools.partial(
        pl.kernel,
        out_type=jax.ShapeDtypeStruct((_N * 32,), jnp.float32),
        mesh=plsc.VectorSubcoreMesh(
            core_axis_name="c", subcore_axis_name="s", num_cores=2, num_subcores=16
        ),
        compiler_params=pltpu.CompilerParams(
            needs_layout_passes=False, use_tc_tiling_on_sc=False
        ),
        scratch_types=_sc_scratch_types(),
    )(_sc_body)


def _sc_scratch_types():
    return [
        pltpu.VMEM((2, 2 * _B), jnp.float32),
        pltpu.VMEM((64, 128), jnp.int32),
        pltpu.VMEM((4096,), jnp.float32),
        pltpu.VMEM((8192, 8), jnp.float32),
        pltpu.VMEM((_B * 32,), jnp.float32),
        pltpu.SemaphoreType.DMA,
        pltpu.SemaphoreType.DMA,
        pltpu.SemaphoreType.DMA,
        pltpu.SemaphoreType.DMA,
    ]


def kernel(x, t, table):
    n = x.shape[0]
    idx = t[0] * (_TIME_RES - 1)
    idx1 = jnp.floor(idx).astype(jnp.int32)
    idx2 = jnp.ceil(idx).astype(jnp.int32)
    w2 = idx - idx1.astype(idx.dtype)
    w1 = 1.0 - w2
    wb = jnp.concatenate(
        [jnp.broadcast_to(w1, (16,)), jnp.broadcast_to(w2, (16,))]
    )
    tn = (
        table.reshape(_TIME_RES, _N_LEVELS, 512, 128, 4)
        .transpose(0, 1, 2, 4, 3)
        .reshape(_TIME_RES * _N_LEVELS * _HASHMAP * 4)
    )
    sel = jnp.zeros((16,), jnp.int32).at[0].set(idx1).at[1].set(idx2)
    c = _get_sc_combine()(tn, sel, wb).reshape(_N_LEVELS * _HASHMAP, 8)
    xbi = x.reshape(8192, 128, 2).transpose(0, 2, 1).reshape(2 * n)
    out = _get_sc_encode()(c, xbi)
    out = out.reshape(4, 8192, 8, 128).transpose(1, 3, 0, 2).reshape(n, 32)
    return out


# speedup vs baseline: 1.1767x; 1.1767x over previous
import functools

import numpy as np
import jax
import jax.numpy as jnp
from jax import lax
from jax.experimental import pallas as pl
from jax.experimental.pallas import tpu as pltpu
from jax.experimental.pallas import tpu_sc as plsc

_TIME_RES = 25
_N_LEVELS = 8
_F = 4
_BASE_RES = 512
_MAX_RES = 32768
_HASHMAP = 1 << 16
_PER_LEVEL_SCALE = float(np.exp2(np.log2(_MAX_RES / _BASE_RES) / (_N_LEVELS - 1)))
_PRIME = np.uint32(2654435761).astype(np.int32)
_SCALES = [np.float32(_BASE_RES * (_PER_LEVEL_SCALE ** l) - 1.0) for l in range(_N_LEVELS)]

_N = 1048576
_NW = 32
_PTS = _N // _NW
_B = 1024
_NCH = _PTS // _B
_GROUPS_A = _B // 16
_GROUPS_B = _B // 4


def _combine_sc_body(tn_hbm, sel_hbm, wb_hbm, c_hbm, t1_v, t2_v, o_v, w_v, sel_v):
    cid = lax.axis_index("c")
    sid = lax.axis_index("s")
    wid = sid * 2 + cid

    i16 = lax.iota(jnp.int32, 16)
    i8 = i16 * 8
    zero16 = jnp.zeros((16,), jnp.float32)

    pltpu.sync_copy(wb_hbm, w_v)
    pltpu.sync_copy(sel_hbm, sel_v)
    w1v = w_v[pl.ds(0, 16)]
    w2v = w_v[pl.ds(16, 16)]
    selvec = sel_v[pl.ds(0, 16)]
    sbase1 = selvec[0] * 2097152
    sbase2 = selvec[1] * 2097152

    def z(g, _):
        o_v[pl.ds(g * 16, 16)] = zero16
        return _

    lax.fori_loop(0, 1024, z, None)

    def chunk(ch, _):
        base = wid * 65536 + ch * 8192
        pltpu.sync_copy(tn_hbm.at[pl.ds(sbase1 + base, 8192)], t1_v)
        pltpu.sync_copy(tn_hbm.at[pl.ds(sbase2 + base, 8192)], t2_v)

        def blk(k, _):
            ib = (k >> 3) * 512 + (k & 7) * 16
            ob = (k >> 3) * 1024 + (k & 7) * 128
            for f in range(4):
                v = w1v * t1_v[pl.ds(ib + f * 128, 16)] + w2v * t2_v[pl.ds(ib + f * 128, 16)]
                plsc.store_scatter(o_v, [ob + i8 + f], v)
            return _

        lax.fori_loop(0, 128, blk, None)
        pltpu.sync_copy(o_v, c_hbm.at[pl.ds(wid * 131072 + ch * 16384, 16384)])
        return _

    lax.fori_loop(0, 8, chunk, None)


@functools.cache
def _get_sc_combine():
    return functools.partial(
        pl.kernel,
        out_type=jax.ShapeDtypeStruct((_N_LEVELS * _HASHMAP * 8,), jnp.float32),
        mesh=plsc.VectorSubcoreMesh(
            core_axis_name="c", subcore_axis_name="s", num_cores=2, num_subcores=16
        ),
        compiler_params=pltpu.CompilerParams(
            needs_layout_passes=False, use_tc_tiling_on_sc=False
        ),
        scratch_types=[
            pltpu.VMEM((8192,), jnp.float32),
            pltpu.VMEM((8192,), jnp.float32),
            pltpu.VMEM((16384,), jnp.float32),
            pltpu.VMEM((32,), jnp.float32),
            pltpu.VMEM((16,), jnp.int32),
        ],
    )(_combine_sc_body)


def _sc_body(
    c_hbm, x_hbm, out_hbm, x_v, ih_v, f_v, rows_v, out_v, sem0, sem1, xsem, osem
):
    cid = lax.axis_index("c")
    sid = lax.axis_index("s")
    wid = sid * 2 + cid
    base_pt = wid * _PTS

    i16 = lax.iota(jnp.int32, 16)
    p4 = i16 >> 2
    c4 = i16 & 3
    pscat = c4 * 128 + p4

    sems = (sem0, sem1)

    def x_dma(ci, xp):
        return pltpu.make_async_copy(
            x_hbm.at[pl.ds((base_pt + ci * _B) * 2, _B * 2)], x_v.at[xp], xsem
        )

    def out_dma(ci, fg):
        px0 = base_pt + ci * _B
        return pltpu.make_async_copy(
            out_v.at[pl.ds(fg * 8192, 8192)],
            out_hbm.at[pl.ds(fg * 8388608 + px0 * 8, 8192)],
            osem,
        )

    def pass_a(lvl, parity, xp):
        scale = _SCALES[lvl]
        fbase = parity * 2048
        ihbase = parity * 32
        lb = np.int32(lvl * _HASHMAP)

        def ga(g, _):
            xb = (g >> 3) * 256 + (g & 7) * 16
            xs = x_v[xp, pl.ds(xb, 16)]
            ys = x_v[xp, pl.ds(xb + 128, 16)]
            pxf = xs * scale + 0.5
            pyf = ys * scale + 0.5
            pxi = pxf.astype(jnp.int32)
            pyi = pyf.astype(jnp.int32)
            f_v[pl.ds(fbase + g * 16, 16)] = pxf - pxi.astype(jnp.float32)
            f_v[pl.ds(fbase + 1024 + g * 16, 16)] = pyf - pyi.astype(jnp.float32)
            m0 = pyi * _PRIME
            m1 = m0 + _PRIME
            px1 = pxi + 1
            row = ihbase + (g >> 3)
            col = (g & 7) * 16
            ih_v[row, pl.ds(col, 16)] = ((pxi ^ m0) & 65535) + lb
            ih_v[row + 8, pl.ds(col, 16)] = ((px1 ^ m0) & 65535) + lb
            ih_v[row + 16, pl.ds(col, 16)] = ((pxi ^ m1) & 65535) + lb
            ih_v[row + 24, pl.ds(col, 16)] = ((px1 ^ m1) & 65535) + lb
            return _

        lax.fori_loop(0, _GROUPS_A, ga, None)

    def fire(parity):
        sem = sems[parity]

        def fj(r, _):
            src = c_hbm.at[ih_v.at[parity * 32 + r]]
            dst = rows_v.at[pl.ds(parity * 4096 + r * 128, 128)]
            pltpu.async_copy(src, dst, sem)
            return _

        lax.fori_loop(0, 32, fj, None)

    def drain(parity):
        sem = sems[parity]

        def dj(r, _):
            src = c_hbm.at[ih_v.at[parity * 32 + r]]
            dst = rows_v.at[pl.ds(parity * 4096 + r * 128, 128)]
            pltpu.make_async_copy(src, dst, sem).wait()
            return _

        lax.fori_loop(0, 32, dj, None)

    def pass_b(lvl, parity):
        rbase = parity * 4096
        fbase = parity * 2048
        obase = np.int32((lvl >> 1) * 8192 + (lvl & 1) * 512)

        def gb(q, _):
            pb = q * 4 + p4
            fxe = plsc.load_gather(f_v, [fbase + pb])
            fye = plsc.load_gather(f_v, [fbase + 1024 + pb])
            gx0 = 1.0 - fxe
            gy0 = 1.0 - fye
            r0 = plsc.load_gather(rows_v, [rbase + pb, c4])
            r1 = plsc.load_gather(rows_v, [rbase + 1024 + pb, c4])
            r2 = plsc.load_gather(rows_v, [rbase + 2048 + pb, c4])
            r3 = plsc.load_gather(rows_v, [rbase + 3072 + pb, c4])
            acc = gy0 * (gx0 * r0 + fxe * r1) + fye * (gx0 * r2 + fxe * r3)
            sbase = obase + (q >> 5) * 1024 + (q & 31) * 4
            plsc.store_scatter(out_v, [sbase + pscat], acc)
            return _

        lax.fori_loop(0, _GROUPS_B, gb, None)

    def chunk(ci, _):
        xp = ci & 1
        x_dma(ci, xp).wait()

        @pl.when(ci < _NCH - 1)
        def _():
            x_dma(ci + 1, 1 - xp).start()

        pass_a(0, 0, xp)
        fire(0)
        pass_a(1, 1, xp)
        fire(1)

        @pl.when(ci > 0)
        def _():
            for fg in range(4):
                out_dma(ci - 1, fg).wait()

        drain(0)
        pass_b(0, 0)
        for lvl in range(2, _N_LEVELS):
            pass_a(lvl, lvl % 2, xp)
            fire(lvl % 2)
            drain((lvl - 1) % 2)
            pass_b(lvl - 1, (lvl - 1) % 2)
        drain((_N_LEVELS - 1) % 2)
        pass_b(_N_LEVELS - 1, (_N_LEVELS - 1) % 2)
        for fg in range(4):
            out_dma(ci, fg).start()
        return _

    x_dma(0, 0).start()
    lax.fori_loop(0, _NCH, chunk, None)
    for fg in range(4):
        out_dma(_NCH - 1, fg).wait()


@functools.cache
def _get_sc_encode():
    return functools.partial(
        pl.kernel,
        out_type=jax.ShapeDtypeStruct((_N * 32,), jnp.float32),
        mesh=plsc.VectorSubcoreMesh(
            core_axis_name="c", subcore_axis_name="s", num_cores=2, num_subcores=16
        ),
        compiler_params=pltpu.CompilerParams(
            needs_layout_passes=False, use_tc_tiling_on_sc=False
        ),
        scratch_types=_sc_scratch_types(),
    )(_sc_body)


def _sc_scratch_types():
    return [
        pltpu.VMEM((2, 2 * _B), jnp.float32),
        pltpu.VMEM((64, 128), jnp.int32),
        pltpu.VMEM((4096,), jnp.float32),
        pltpu.VMEM((8192, 8), jnp.float32),
        pltpu.VMEM((_B * 32,), jnp.float32),
        pltpu.SemaphoreType.DMA,
        pltpu.SemaphoreType.DMA,
        pltpu.SemaphoreType.DMA,
        pltpu.SemaphoreType.DMA,
    ]


def kernel(x, t, table):
    n = x.shape[0]
    idx = t[0] * (_TIME_RES - 1)
    idx1 = jnp.floor(idx).astype(jnp.int32)
    idx2 = jnp.ceil(idx).astype(jnp.int32)
    w2 = idx - idx1.astype(idx.dtype)
    w1 = 1.0 - w2
    wb = jnp.concatenate(
        [jnp.broadcast_to(w1, (16,)), jnp.broadcast_to(w2, (16,))]
    )
    tn = (
        table.reshape(_TIME_RES, _N_LEVELS, 512, 128, 4)
        .transpose(0, 1, 2, 4, 3)
        .reshape(_TIME_RES * _N_LEVELS * _HASHMAP * 4)
    )
    sel = jnp.zeros((16,), jnp.int32).at[0].set(idx1).at[1].set(idx2)
    c = _get_sc_combine()(tn, sel, wb).reshape(_N_LEVELS * _HASHMAP, 8)
    xbi = x.reshape(8192, 128, 2).transpose(0, 2, 1).reshape(2 * n)
    out = _get_sc_encode()(c, xbi)
    out = out.reshape(4, 8192, 8, 128).transpose(1, 3, 0, 2).reshape(n, 32)
    return out
